# baseline (speedup 1.0000x reference)
import dataclasses
import functools

import jax
import jax.numpy as jnp
from jax import lax
from jax.experimental import pallas as pl
from jax.experimental.pallas import tpu as pltpu
from jax.experimental.pallas import tpu_sc as plsc

N = 10000
E = 320000
NF = 128
NH = 128

NC = 2
NS = 16
NT = NC * NS
EW = E // NT
CH = 80
NCH = EW // CH
FH = NH // NC
EW2 = E // NS
NCH2 = EW2 // CH
RPT0 = 624
TAILO = NS * RPT0
TAILR = N - TAILO
ZBR = 208

BLK = 1000
GRID = N // BLK

@functools.cache
def _sc_kernels():
    mesh = plsc.VectorSubcoreMesh(core_axis_name="c", subcore_axis_name="s")
    cp = pltpu.CompilerParams()
    if "needs_layout_passes" in pltpu.CompilerParams.__dataclass_fields__:
        cp = dataclasses.replace(cp, needs_layout_passes=False)
    if "use_tc_tiling_on_sc" in pltpu.CompilerParams.__dataclass_fields__:
        cp = dataclasses.replace(cp, use_tc_tiling_on_sc=False)
    deg = functools.partial(
        pl.kernel,
        compiler_params=cp,
        out_type=jax.ShapeDtypeStruct((NC, N, 16), jnp.float32),
        mesh=mesh,
        scratch_types=[
            pltpu.VMEM((NCH, CH), jnp.float32),
            pltpu.VMEM((NCH, CH), jnp.int32),
            pltpu.VMEM((CH, 16), jnp.float32),
            pltpu.VMEM((CH, 16), jnp.float32),
            pltpu.VMEM((ZBR, 16), jnp.float32),
            pltpu.VMEM_SHARED((N, 16), jnp.float32),
            pltpu.SemaphoreType.DMA,
            pltpu.SemaphoreType.DMA,
        ],
    )(_deg_body)
    scat = functools.partial(
        pl.kernel,
        compiler_params=cp,
        out_type=jax.ShapeDtypeStruct((NC, N, FH), jnp.float32),
        mesh=mesh,
        scratch_types=[
            pltpu.VMEM((NCH2, CH), jnp.int32),
            pltpu.VMEM((NCH2, CH), jnp.float32),
            pltpu.VMEM((NCH2, CH), jnp.int32),
            pltpu.VMEM((CH, FH), jnp.float32),
            pltpu.VMEM((CH, FH), jnp.float32),
            pltpu.VMEM((ZBR, FH), jnp.float32),
            pltpu.VMEM_SHARED((N, FH), jnp.float32),
            pltpu.SemaphoreType.DMA,
            pltpu.SemaphoreType.DMA,
            pltpu.SemaphoreType.DMA,
            pltpu.SemaphoreType.DMA,
        ],
    )(_scatter_body)
    return deg, scat


_GDN = lax.GatherDimensionNumbers(offset_dims=(), collapsed_slice_dims=(0,),
                                  start_index_map=(0,))


def _bcast16(vec, k):
    idx = jnp.full((16, 1), k, jnp.int32)
    return lax.gather(vec, idx, _GDN, (1,),
                      mode=lax.GatherScatterMode.PROMISE_IN_BOUNDS)


def _zero_fill(zb, acc, sid, ncol):
    @pl.loop(0, ZBR)
    def _(r):
        for f in range(ncol // 16):
            zb[r, pl.ds(f * 16, 16)] = jnp.zeros((16,), jnp.float32)

    @pl.loop(0, RPT0 // ZBR)
    def _(k):
        pltpu.sync_copy(zb, acc.at[pl.ds(sid * RPT0 + k * ZBR, ZBR)])

    @pl.when(sid == 0)
    def _():
        pltpu.sync_copy(zb.at[pl.ds(0, TAILR)], acc.at[pl.ds(TAILO, TAILR)])


def _write_out(acc, out2d, sid):
    pltpu.sync_copy(acc.at[pl.ds(sid * RPT0, RPT0)],
                    out2d.at[pl.ds(sid * RPT0, RPT0)])

    @pl.when(sid == 0)
    def _():
        pltpu.sync_copy(acc.at[pl.ds(TAILO, TAILR)],
                        out2d.at[pl.ds(TAILO, TAILR)])


def _deg_body(dst_hbm, w_hbm, out_hbm, w_v, dst_v, val0, val1, zb, acc,
              ssem0, ssem1):
    cid = lax.axis_index("c")
    sid = lax.axis_index("s")
    tid = cid * NS + sid

    pltpu.sync_copy(w_hbm.at[tid], w_v)
    pltpu.sync_copy(dst_hbm.at[tid], dst_v)

    _zero_fill(zb, acc, sid, 16)
    plsc.subcore_barrier()

    bufs = ((val0, ssem0), (val1, ssem1))

    def _fill(val, j):
        @pl.loop(0, CH // 16)
        def _(g):
            w16 = w_v[j, pl.ds(g * 16, 16)]
            for k in range(16):
                val[g * 16 + k, pl.ds(0, 16)] = _bcast16(w16, k)

    @pl.loop(0, NCH - 1, step=2)
    def _(j):
        for p in range(2):
            val, ssem = bufs[p]
            c = j + p

            @pl.when(c >= 2)
            def _():
                pltpu.make_async_copy(val, acc.at[dst_v.at[c - 2]],
                                      ssem).wait()

            _fill(val, c)
            pltpu.async_copy(val, acc.at[dst_v.at[c]], ssem, add=True)

    pltpu.make_async_copy(val0, acc.at[dst_v.at[NCH - 3]], ssem0).wait()
    _fill(val0, NCH - 1)
    pltpu.async_copy(val0, acc.at[dst_v.at[NCH - 1]], ssem0, add=True)

    pltpu.make_async_copy(val0, acc.at[dst_v.at[NCH - 1]], ssem0).wait()
    pltpu.make_async_copy(val1, acc.at[dst_v.at[NCH - 2]], ssem1).wait()
    plsc.subcore_barrier()
    _write_out(acc, out_hbm.at[cid], sid)


def _scatter_body(ys_hbm, src_hbm, dst_hbm, w_hbm, out_hbm,
                  src_v, w_v, dst_v, rows0, rows1, zb, acc,
                  gsem0, gsem1, ssem0, ssem1):
    cid = lax.axis_index("c")
    sid = lax.axis_index("s")

    pltpu.sync_copy(src_hbm.at[sid], src_v)
    pltpu.sync_copy(w_hbm.at[sid], w_v)
    pltpu.sync_copy(dst_hbm.at[sid], dst_v)

    _zero_fill(zb, acc, sid, FH)
    plsc.subcore_barrier()

    ys2d = ys_hbm.at[cid]
    bufs = ((rows0, gsem0, ssem0), (rows1, gsem1, ssem1))

    pltpu.async_copy(ys2d.at[src_v.at[0]], rows0, gsem0)

    @pl.loop(0, NCH2, step=2)
    def _(j):
        for p in range(2):
            rows, gsem, ssem = bufs[p]
            orows, ogsem, ossem = bufs[1 - p]
            c = j + p

            @pl.when(c >= 1)
            def _():
                pltpu.make_async_copy(orows, acc.at[dst_v.at[c - 1]],
                                      ossem).wait()

            @pl.when(c + 1 < NCH2)
            def _():
                pltpu.async_copy(ys2d.at[src_v.at[c + 1]], orows, ogsem)

            pltpu.make_async_copy(ys2d.at[src_v.at[c]], rows, gsem).wait()

            @pl.loop(0, CH // 16)
            def _(g):
                w16 = w_v[c, pl.ds(g * 16, 16)]
                for k in range(16):
                    b = _bcast16(w16, k)
                    e = g * 16 + k
                    for f in range(FH // 16):
                        sl = (e, pl.ds(f * 16, 16))
                        rows[sl] = rows[sl] * b

            pltpu.async_copy(rows, acc.at[dst_v.at[c]], ssem, add=True)

    pltpu.make_async_copy(rows1, acc.at[dst_v.at[NCH2 - 1]], ssem1).wait()
    plsc.subcore_barrier()
    _write_out(acc, out_hbm.at[cid], sid)


def _mm_sum_body(x_ref, w_ref, xw_ref, xsum_ref):
    i = pl.program_id(0)
    xb = x_ref[...]
    xw_ref[...] = jnp.dot(xb, w_ref[...], preferred_element_type=jnp.float32)

    @pl.when(i == 0)
    def _():
        xsum_ref[...] = jnp.zeros_like(xsum_ref)

    xsum_ref[...] += jnp.sum(xb, axis=0, keepdims=True)


def _mm_sum(x, w):
    return pl.pallas_call(
        _mm_sum_body,
        grid=(GRID,),
        in_specs=[pl.BlockSpec((BLK, NF), lambda i: (i, 0)),
                  pl.BlockSpec((NF, NH), lambda i: (0, 0))],
        out_specs=[pl.BlockSpec((BLK, NH), lambda i: (i, 0)),
                   pl.BlockSpec((1, NF), lambda i: (0, 0))],
        out_shape=[jax.ShapeDtypeStruct((N, NH), jnp.float32),
                   jax.ShapeDtypeStruct((1, NF), jnp.float32)],
    )(x, w)


def _dis_body(degp_ref, xw_ref, ys_ref, disr_ref):
    d = degp_ref[0] + degp_ref[1]
    dis = lax.rsqrt(d[:, 0:1] + 1.0)
    disb = jnp.broadcast_to(dis, (BLK, NH))
    disr_ref[...] = disb
    ys = xw_ref[...] * disb
    ys_ref[...] = jnp.stack([ys[:, :FH], ys[:, FH:]], axis=0)


def _dis_scale(degp, xw):
    return pl.pallas_call(
        _dis_body,
        grid=(GRID,),
        in_specs=[pl.BlockSpec((NC, BLK, 16), lambda i: (0, i, 0)),
                  pl.BlockSpec((BLK, NH), lambda i: (i, 0))],
        out_specs=[pl.BlockSpec((NC, BLK, FH), lambda i: (0, i, 0)),
                   pl.BlockSpec((BLK, NH), lambda i: (i, 0))],
        out_shape=[jax.ShapeDtypeStruct((NC, N, FH), jnp.float32),
                   jax.ShapeDtypeStruct((N, NH), jnp.float32)],
    )(degp, xw)


def _gcn_post_body(part_ref, ys_ref, disr_ref, b_ref, h_ref, st_ref):
    i = pl.program_id(0)
    acc = (jnp.concatenate([part_ref[0], part_ref[1]], axis=1)
           + jnp.concatenate([ys_ref[0], ys_ref[1]], axis=1))
    h = jnp.maximum(disr_ref[...] * acc + b_ref[...], 0.0)
    h_ref[...] = h

    @pl.when(i == 0)
    def _():
        st_ref[...] = jnp.zeros_like(st_ref)

    st_ref[0:1, :] += jnp.sum(h, axis=0, keepdims=True)
    st_ref[1:2, :] += jnp.sum(h * h, axis=0, keepdims=True)


def _gcn_post(part, ys, disr, b):
    return pl.pallas_call(
        _gcn_post_body,
        grid=(GRID,),
        in_specs=[pl.BlockSpec((NC, BLK, FH), lambda i: (0, i, 0)),
                  pl.BlockSpec((NC, BLK, FH), lambda i: (0, i, 0)),
                  pl.BlockSpec((BLK, NH), lambda i: (i, 0)),
                  pl.BlockSpec((1, NH), lambda i: (0, 0))],
        out_specs=[pl.BlockSpec((BLK, NH), lambda i: (i, 0)),
                   pl.BlockSpec((2, NH), lambda i: (0, 0))],
        out_shape=[jax.ShapeDtypeStruct((N, NH), jnp.float32),
                   jax.ShapeDtypeStruct((2, NH), jnp.float32)],
    )(part, ys, disr, b)


def _bn_mm_body(h_ref, st_ref, g_ref, be_ref, w_ref, disr_ref,
                hn_ref, ys2_ref):
    m = st_ref[0:1, :] / N
    v = st_ref[1:2, :] / N - m * m
    hn = (h_ref[...] - m) * lax.rsqrt(v + 1e-5) * g_ref[...] + be_ref[...]
    hn_ref[...] = hn
    xw2 = jnp.dot(hn, w_ref[...], preferred_element_type=jnp.float32)
    ys2 = xw2 * disr_ref[...]
    ys2_ref[...] = jnp.stack([ys2[:, :FH], ys2[:, FH:]], axis=0)


def _bn_mm(h, st, g, be, w, disr):
    return pl.pallas_call(
        _bn_mm_body,
        grid=(GRID,),
        in_specs=[pl.BlockSpec((BLK, NH), lambda i: (i, 0)),
                  pl.BlockSpec((2, NH), lambda i: (0, 0)),
                  pl.BlockSpec((1, NH), lambda i: (0, 0)),
                  pl.BlockSpec((1, NH), lambda i: (0, 0)),
                  pl.BlockSpec((NH, NH), lambda i: (0, 0)),
                  pl.BlockSpec((BLK, NH), lambda i: (i, 0))],
        out_specs=[pl.BlockSpec((BLK, NH), lambda i: (i, 0)),
                   pl.BlockSpec((NC, BLK, FH), lambda i: (0, i, 0))],
        out_shape=[jax.ShapeDtypeStruct((N, NH), jnp.float32),
                   jax.ShapeDtypeStruct((NC, N, FH), jnp.float32)],
    )(h, st, g, be, w, disr)


def _head_body(h1n_ref, h2_ref, st2_ref, g2_ref, be2_ref,
               wih1t_ref, bi1_ref, wih2t_ref, bi2_ref,
               wfc1_ref, bfc1_ref, wfc2_ref, bfc2_ref, xmean_ref, o_ref):
    m = st2_ref[0:1, :] / N
    v = st2_ref[1:2, :] / N - m * m
    h2n = ((h2_ref[...] - m) * lax.rsqrt(v + 1e-5) * g2_ref[...]
           + be2_ref[...])
    hc = jnp.concatenate([h1n_ref[...], h2n], axis=1)
    ga = jnp.dot(hc, wih1t_ref[...],
                 preferred_element_type=jnp.float32) + bi1_ref[...]
    i1 = jax.nn.sigmoid(ga[:, 0:NH])
    g1 = jnp.tanh(ga[:, NH:2 * NH])
    o1 = jax.nn.sigmoid(ga[:, 2 * NH:3 * NH])
    hn1 = o1 * jnp.tanh(i1 * g1)
    gb = jnp.dot(hn1, wih2t_ref[...],
                 preferred_element_type=jnp.float32) + bi2_ref[...]
    i2 = jax.nn.sigmoid(gb[:, 0:NH])
    g2g = jnp.tanh(gb[:, NH:2 * NH])
    o2 = jax.nn.sigmoid(gb[:, 2 * NH:3 * NH])
    hn2 = o2 * jnp.tanh(i2 * g2g)
    z = jnp.concatenate(
        [hn1, hn2, jnp.broadcast_to(xmean_ref[...], (BLK, NF))], axis=1)
    a = jnp.maximum(
        jnp.dot(z, wfc1_ref[...], preferred_element_type=jnp.float32)
        + bfc1_ref[...], 0.0)
    o_ref[...] = (jnp.dot(a, wfc2_ref[...],
                          preferred_element_type=jnp.float32)
                  + bfc2_ref[...])


def _head(h1n, h2, st2, g2, be2, wih1t, bi1, wih2t, bi2,
          wfc1, bfc1, wfc2, bfc2, xmean):
    return pl.pallas_call(
        _head_body,
        grid=(GRID,),
        in_specs=[pl.BlockSpec((BLK, NH), lambda i: (i, 0)),
                  pl.BlockSpec((BLK, NH), lambda i: (i, 0)),
                  pl.BlockSpec((2, NH), lambda i: (0, 0)),
                  pl.BlockSpec((1, NH), lambda i: (0, 0)),
                  pl.BlockSpec((1, NH), lambda i: (0, 0)),
                  pl.BlockSpec((2 * NH, 3 * NH), lambda i: (0, 0)),
                  pl.BlockSpec((1, 3 * NH), lambda i: (0, 0)),
                  pl.BlockSpec((NH, 3 * NH), lambda i: (0, 0)),
                  pl.BlockSpec((1, 3 * NH), lambda i: (0, 0)),
                  pl.BlockSpec((2 * NH + NF, NH), lambda i: (0, 0)),
                  pl.BlockSpec((1, NH), lambda i: (0, 0)),
                  pl.BlockSpec((NH, 1), lambda i: (0, 0)),
                  pl.BlockSpec((1, 1), lambda i: (0, 0)),
                  pl.BlockSpec((1, NF), lambda i: (0, 0))],
        out_specs=pl.BlockSpec((BLK, 1), lambda i: (i, 0)),
        out_shape=jax.ShapeDtypeStruct((N, 1), jnp.float32),
    )(h1n, h2, st2, g2, be2, wih1t, bi1, wih2t, bi2,
      wfc1, bfc1, wfc2, bfc2, xmean)


def kernel(x, edge_index, edge_attr, W1, b1, g1, be1, W2, b2, g2, be2,
           Wih1, Whh1, bih1, bhh1, Wih2, Whh2, bih2, bhh2,
           Wfc1, bfc1, Wfc2, bfc2):
    src = edge_index[0].reshape(NS, NCH2, CH)
    dst16 = edge_index[1].reshape(NS, NCH2, CH)
    ew16 = edge_attr.reshape(NS, NCH2, CH)
    dst32 = edge_index[1].reshape(NT, NCH, CH)
    ew32 = edge_attr.reshape(NT, NCH, CH)

    def _prep(wih, bih, bhh):
        wt = wih.T
        wt = jnp.concatenate([wt[:, 0:NH], wt[:, 2 * NH:4 * NH]], axis=1)
        bb = (bih + bhh)
        bb = jnp.concatenate([bb[0:NH], bb[2 * NH:4 * NH]]).reshape(1, 3 * NH)
        return wt, bb

    wih1t, bi1 = _prep(Wih1, bih1, bhh1)
    wih2t, bi2 = _prep(Wih2, bih2, bhh2)

    _deg_kernel, _edge_scatter = _sc_kernels()
    degp = _deg_kernel(dst32, ew32)
    xw1, xsum = _mm_sum(x, W1)
    ys1, disr = _dis_scale(degp, xw1)
    part1 = _edge_scatter(ys1, src, dst16, ew16)
    h1, st1 = _gcn_post(part1, ys1, disr, b1.reshape(1, NH))
    h1n, ys2 = _bn_mm(h1, st1, g1.reshape(1, NH), be1.reshape(1, NH),
                      W2, disr)
    part2 = _edge_scatter(ys2, src, dst16, ew16)
    h2, st2 = _gcn_post(part2, ys2, disr, b2.reshape(1, NH))
    out = _head(h1n, h2, st2, g2.reshape(1, NH), be2.reshape(1, NH),
                wih1t, bi1, wih2t, bi2,
                Wfc1, bfc1.reshape(1, NH), Wfc2, bfc2.reshape(1, 1),
                xsum / N)
    return out.reshape(N)


# speedup vs baseline: 1.5796x; 1.5796x over previous
import dataclasses
import functools

import jax
import jax.numpy as jnp
from jax import lax
from jax.experimental import pallas as pl
from jax.experimental.pallas import tpu as pltpu
from jax.experimental.pallas import tpu_sc as plsc

N = 10000
E = 320000
NF = 128
NH = 128

NC = 2
NS = 16
NT = NC * NS
EW = E // NT
CH = 80
NCH = EW // CH
FH = NH // NC
EW2 = E // NS
NCH2 = EW2 // CH
RPT0 = 624
TAILO = NS * RPT0
TAILR = N - TAILO
ZBR = 208

BLK = 1000
GRID = N // BLK

@functools.cache
def _sc_kernels():
    mesh = plsc.VectorSubcoreMesh(core_axis_name="c", subcore_axis_name="s")
    cp = pltpu.CompilerParams()
    if "needs_layout_passes" in pltpu.CompilerParams.__dataclass_fields__:
        cp = dataclasses.replace(cp, needs_layout_passes=False)
    if "use_tc_tiling_on_sc" in pltpu.CompilerParams.__dataclass_fields__:
        cp = dataclasses.replace(cp, use_tc_tiling_on_sc=False)
    deg = functools.partial(
        pl.kernel,
        compiler_params=cp,
        out_type=jax.ShapeDtypeStruct((NC, N, 16), jnp.float32),
        mesh=mesh,
        scratch_types=[
            pltpu.VMEM((NCH, CH), jnp.float32),
            pltpu.VMEM((NCH, CH), jnp.int32),
            pltpu.VMEM((CH, 16), jnp.float32),
            pltpu.VMEM((CH, 16), jnp.float32),
            pltpu.VMEM((ZBR, 16), jnp.float32),
            pltpu.VMEM_SHARED((N, 16), jnp.float32),
            pltpu.SemaphoreType.DMA,
            pltpu.SemaphoreType.DMA,
        ],
    )(_deg_body)
    scat = functools.partial(
        pl.kernel,
        compiler_params=cp,
        out_type=jax.ShapeDtypeStruct((NC, N, FH), jnp.float32),
        mesh=mesh,
        scratch_types=[
            pltpu.VMEM((NCH2, CH), jnp.int32),
            pltpu.VMEM((NCH2, CH), jnp.float32),
            pltpu.VMEM((NCH2, CH), jnp.int32),
            pltpu.VMEM((CH, FH), jnp.float32),
            pltpu.VMEM((CH, FH), jnp.float32),
            pltpu.VMEM((ZBR, FH), jnp.float32),
            pltpu.VMEM_SHARED((N, FH), jnp.float32),
            pltpu.SemaphoreType.DMA,
            pltpu.SemaphoreType.DMA,
            pltpu.SemaphoreType.DMA,
            pltpu.SemaphoreType.DMA,
        ],
    )(_scatter_body)
    return deg, scat


_GDN = lax.GatherDimensionNumbers(offset_dims=(), collapsed_slice_dims=(0,),
                                  start_index_map=(0,))


def _bcast16(vec, k):
    idx = jnp.full((16, 1), k, jnp.int32)
    return lax.gather(vec, idx, _GDN, (1,),
                      mode=lax.GatherScatterMode.PROMISE_IN_BOUNDS)


def _zero_fill(zb, acc, sid, ncol):
    @pl.loop(0, ZBR)
    def _(r):
        for f in range(ncol // 16):
            zb[r, pl.ds(f * 16, 16)] = jnp.zeros((16,), jnp.float32)

    @pl.loop(0, RPT0 // ZBR)
    def _(k):
        pltpu.sync_copy(zb, acc.at[pl.ds(sid * RPT0 + k * ZBR, ZBR)])

    @pl.when(sid == 0)
    def _():
        pltpu.sync_copy(zb.at[pl.ds(0, TAILR)], acc.at[pl.ds(TAILO, TAILR)])


def _write_out(acc, out2d, sid):
    pltpu.sync_copy(acc.at[pl.ds(sid * RPT0, RPT0)],
                    out2d.at[pl.ds(sid * RPT0, RPT0)])

    @pl.when(sid == 0)
    def _():
        pltpu.sync_copy(acc.at[pl.ds(TAILO, TAILR)],
                        out2d.at[pl.ds(TAILO, TAILR)])


def _deg_body(dst_hbm, w_hbm, out_hbm, w_v, dst_v, val0, val1, zb, acc,
              ssem0, ssem1):
    cid = lax.axis_index("c")
    sid = lax.axis_index("s")
    tid = cid * NS + sid

    pltpu.sync_copy(w_hbm.at[tid], w_v)
    pltpu.sync_copy(dst_hbm.at[tid], dst_v)

    _zero_fill(zb, acc, sid, 16)
    plsc.subcore_barrier()

    bufs = ((val0, ssem0), (val1, ssem1))

    def _fill(val, j):
        @pl.loop(0, CH // 16)
        def _(g):
            w16 = w_v[j, pl.ds(g * 16, 16)]
            for k in range(16):
                val[g * 16 + k, pl.ds(0, 16)] = _bcast16(w16, k)

    @pl.loop(0, NCH - 1, step=2)
    def _(j):
        for p in range(2):
            val, ssem = bufs[p]
            c = j + p

            @pl.when(c >= 2)
            def _():
                pltpu.make_async_copy(val, acc.at[dst_v.at[c - 2]],
                                      ssem).wait()

            _fill(val, c)
            pltpu.async_copy(val, acc.at[dst_v.at[c]], ssem, add=True)

    pltpu.make_async_copy(val0, acc.at[dst_v.at[NCH - 3]], ssem0).wait()
    _fill(val0, NCH - 1)
    pltpu.async_copy(val0, acc.at[dst_v.at[NCH - 1]], ssem0, add=True)

    pltpu.make_async_copy(val0, acc.at[dst_v.at[NCH - 1]], ssem0).wait()
    pltpu.make_async_copy(val1, acc.at[dst_v.at[NCH - 2]], ssem1).wait()
    plsc.subcore_barrier()
    _write_out(acc, out_hbm.at[cid], sid)


def _scatter_body(ys_hbm, src_hbm, dst_hbm, w_hbm, out_hbm,
                  src_v, w_v, dst_v, rows0, rows1, zb, acc,
                  gsem0, gsem1, ssem0, ssem1):
    cid = lax.axis_index("c")
    sid = lax.axis_index("s")

    pltpu.sync_copy(src_hbm.at[sid], src_v)
    pltpu.sync_copy(w_hbm.at[sid], w_v)
    pltpu.sync_copy(dst_hbm.at[sid], dst_v)

    _zero_fill(zb, acc, sid, FH)
    plsc.subcore_barrier()

    ys2d = ys_hbm.at[cid]
    bufs = ((rows0, gsem0, ssem0), (rows1, gsem1, ssem1))

    pltpu.async_copy(ys2d.at[src_v.at[0]], rows0, gsem0)

    @pl.loop(0, NCH2, step=2)
    def _(j):
        for p in range(2):
            rows, gsem, ssem = bufs[p]
            orows, ogsem, ossem = bufs[1 - p]
            c = j + p

            @pl.when(c >= 1)
            def _():
                pltpu.make_async_copy(orows, acc.at[dst_v.at[c - 1]],
                                      ossem).wait()

            @pl.when(c + 1 < NCH2)
            def _():
                pltpu.async_copy(ys2d.at[src_v.at[c + 1]], orows, ogsem)

            pltpu.make_async_copy(ys2d.at[src_v.at[c]], rows, gsem).wait()
            jj = jnp.full((16,), c, jnp.int32)

            @pl.loop(0, CH, unroll=4)
            def _(e):
                b = plsc.load_gather(w_v,
                                     [jj, jnp.full((16,), e, jnp.int32)])
                for f in range(FH // 16):
                    sl = (e, pl.ds(f * 16, 16))
                    rows[sl] = rows[sl] * b

            pltpu.async_copy(rows, acc.at[dst_v.at[c]], ssem, add=True)

    pltpu.make_async_copy(rows1, acc.at[dst_v.at[NCH2 - 1]], ssem1).wait()
    plsc.subcore_barrier()
    _write_out(acc, out_hbm.at[cid], sid)


def _mm_sum_body(x_ref, w_ref, xw_ref, xsum_ref):
    i = pl.program_id(0)
    xb = x_ref[...]
    xw_ref[...] = jnp.dot(xb, w_ref[...], preferred_element_type=jnp.float32)

    @pl.when(i == 0)
    def _():
        xsum_ref[...] = jnp.zeros_like(xsum_ref)

    xsum_ref[...] += jnp.sum(xb, axis=0, keepdims=True)


def _mm_sum(x, w):
    return pl.pallas_call(
        _mm_sum_body,
        grid=(GRID,),
        in_specs=[pl.BlockSpec((BLK, NF), lambda i: (i, 0)),
                  pl.BlockSpec((NF, NH), lambda i: (0, 0))],
        out_specs=[pl.BlockSpec((BLK, NH), lambda i: (i, 0)),
                   pl.BlockSpec((1, NF), lambda i: (0, 0))],
        out_shape=[jax.ShapeDtypeStruct((N, NH), jnp.float32),
                   jax.ShapeDtypeStruct((1, NF), jnp.float32)],
    )(x, w)


def _dis_body(degp_ref, xw_ref, ys_ref, disr_ref):
    d = degp_ref[0] + degp_ref[1]
    dis = lax.rsqrt(d[:, 0:1] + 1.0)
    disb = jnp.broadcast_to(dis, (BLK, NH))
    disr_ref[...] = disb
    ys = xw_ref[...] * disb
    ys_ref[...] = jnp.stack([ys[:, :FH], ys[:, FH:]], axis=0)


def _dis_scale(degp, xw):
    return pl.pallas_call(
        _dis_body,
        grid=(GRID,),
        in_specs=[pl.BlockSpec((NC, BLK, 16), lambda i: (0, i, 0)),
                  pl.BlockSpec((BLK, NH), lambda i: (i, 0))],
        out_specs=[pl.BlockSpec((NC, BLK, FH), lambda i: (0, i, 0)),
                   pl.BlockSpec((BLK, NH), lambda i: (i, 0))],
        out_shape=[jax.ShapeDtypeStruct((NC, N, FH), jnp.float32),
                   jax.ShapeDtypeStruct((N, NH), jnp.float32)],
    )(degp, xw)


def _gcn_post_body(part_ref, ys_ref, disr_ref, b_ref, h_ref, st_ref):
    i = pl.program_id(0)
    acc = (jnp.concatenate([part_ref[0], part_ref[1]], axis=1)
           + jnp.concatenate([ys_ref[0], ys_ref[1]], axis=1))
    h = jnp.maximum(disr_ref[...] * acc + b_ref[...], 0.0)
    h_ref[...] = h

    @pl.when(i == 0)
    def _():
        st_ref[...] = jnp.zeros_like(st_ref)

    st_ref[0:1, :] += jnp.sum(h, axis=0, keepdims=True)
    st_ref[1:2, :] += jnp.sum(h * h, axis=0, keepdims=True)


def _gcn_post(part, ys, disr, b):
    return pl.pallas_call(
        _gcn_post_body,
        grid=(GRID,),
        in_specs=[pl.BlockSpec((NC, BLK, FH), lambda i: (0, i, 0)),
                  pl.BlockSpec((NC, BLK, FH), lambda i: (0, i, 0)),
                  pl.BlockSpec((BLK, NH), lambda i: (i, 0)),
                  pl.BlockSpec((1, NH), lambda i: (0, 0))],
        out_specs=[pl.BlockSpec((BLK, NH), lambda i: (i, 0)),
                   pl.BlockSpec((2, NH), lambda i: (0, 0))],
        out_shape=[jax.ShapeDtypeStruct((N, NH), jnp.float32),
                   jax.ShapeDtypeStruct((2, NH), jnp.float32)],
    )(part, ys, disr, b)


def _bn_mm_body(h_ref, st_ref, g_ref, be_ref, w_ref, disr_ref,
                hn_ref, ys2_ref):
    m = st_ref[0:1, :] / N
    v = st_ref[1:2, :] / N - m * m
    hn = (h_ref[...] - m) * lax.rsqrt(v + 1e-5) * g_ref[...] + be_ref[...]
    hn_ref[...] = hn
    xw2 = jnp.dot(hn, w_ref[...], preferred_element_type=jnp.float32)
    ys2 = xw2 * disr_ref[...]
    ys2_ref[...] = jnp.stack([ys2[:, :FH], ys2[:, FH:]], axis=0)


def _bn_mm(h, st, g, be, w, disr):
    return pl.pallas_call(
        _bn_mm_body,
        grid=(GRID,),
        in_specs=[pl.BlockSpec((BLK, NH), lambda i: (i, 0)),
                  pl.BlockSpec((2, NH), lambda i: (0, 0)),
                  pl.BlockSpec((1, NH), lambda i: (0, 0)),
                  pl.BlockSpec((1, NH), lambda i: (0, 0)),
                  pl.BlockSpec((NH, NH), lambda i: (0, 0)),
                  pl.BlockSpec((BLK, NH), lambda i: (i, 0))],
        out_specs=[pl.BlockSpec((BLK, NH), lambda i: (i, 0)),
                   pl.BlockSpec((NC, BLK, FH), lambda i: (0, i, 0))],
        out_shape=[jax.ShapeDtypeStruct((N, NH), jnp.float32),
                   jax.ShapeDtypeStruct((NC, N, FH), jnp.float32)],
    )(h, st, g, be, w, disr)


def _head_body(h1n_ref, h2_ref, st2_ref, g2_ref, be2_ref,
               wih1t_ref, bi1_ref, wih2t_ref, bi2_ref,
               wfc1_ref, bfc1_ref, wfc2_ref, bfc2_ref, xmean_ref, o_ref):
    m = st2_ref[0:1, :] / N
    v = st2_ref[1:2, :] / N - m * m
    h2n = ((h2_ref[...] - m) * lax.rsqrt(v + 1e-5) * g2_ref[...]
           + be2_ref[...])
    hc = jnp.concatenate([h1n_ref[...], h2n], axis=1)
    ga = jnp.dot(hc, wih1t_ref[...],
                 preferred_element_type=jnp.float32) + bi1_ref[...]
    i1 = jax.nn.sigmoid(ga[:, 0:NH])
    g1 = jnp.tanh(ga[:, NH:2 * NH])
    o1 = jax.nn.sigmoid(ga[:, 2 * NH:3 * NH])
    hn1 = o1 * jnp.tanh(i1 * g1)
    gb = jnp.dot(hn1, wih2t_ref[...],
                 preferred_element_type=jnp.float32) + bi2_ref[...]
    i2 = jax.nn.sigmoid(gb[:, 0:NH])
    g2g = jnp.tanh(gb[:, NH:2 * NH])
    o2 = jax.nn.sigmoid(gb[:, 2 * NH:3 * NH])
    hn2 = o2 * jnp.tanh(i2 * g2g)
    z = jnp.concatenate(
        [hn1, hn2, jnp.broadcast_to(xmean_ref[...], (BLK, NF))], axis=1)
    a = jnp.maximum(
        jnp.dot(z, wfc1_ref[...], preferred_element_type=jnp.float32)
        + bfc1_ref[...], 0.0)
    o_ref[...] = (jnp.dot(a, wfc2_ref[...],
                          preferred_element_type=jnp.float32)
                  + bfc2_ref[...])


def _head(h1n, h2, st2, g2, be2, wih1t, bi1, wih2t, bi2,
          wfc1, bfc1, wfc2, bfc2, xmean):
    return pl.pallas_call(
        _head_body,
        grid=(GRID,),
        in_specs=[pl.BlockSpec((BLK, NH), lambda i: (i, 0)),
                  pl.BlockSpec((BLK, NH), lambda i: (i, 0)),
                  pl.BlockSpec((2, NH), lambda i: (0, 0)),
                  pl.BlockSpec((1, NH), lambda i: (0, 0)),
                  pl.BlockSpec((1, NH), lambda i: (0, 0)),
                  pl.BlockSpec((2 * NH, 3 * NH), lambda i: (0, 0)),
                  pl.BlockSpec((1, 3 * NH), lambda i: (0, 0)),
                  pl.BlockSpec((NH, 3 * NH), lambda i: (0, 0)),
                  pl.BlockSpec((1, 3 * NH), lambda i: (0, 0)),
                  pl.BlockSpec((2 * NH + NF, NH), lambda i: (0, 0)),
                  pl.BlockSpec((1, NH), lambda i: (0, 0)),
                  pl.BlockSpec((NH, 1), lambda i: (0, 0)),
                  pl.BlockSpec((1, 1), lambda i: (0, 0)),
                  pl.BlockSpec((1, NF), lambda i: (0, 0))],
        out_specs=pl.BlockSpec((BLK, 1), lambda i: (i, 0)),
        out_shape=jax.ShapeDtypeStruct((N, 1), jnp.float32),
    )(h1n, h2, st2, g2, be2, wih1t, bi1, wih2t, bi2,
      wfc1, bfc1, wfc2, bfc2, xmean)


def kernel(x, edge_index, edge_attr, W1, b1, g1, be1, W2, b2, g2, be2,
           Wih1, Whh1, bih1, bhh1, Wih2, Whh2, bih2, bhh2,
           Wfc1, bfc1, Wfc2, bfc2):
    src = edge_index[0].reshape(NS, NCH2, CH)
    dst16 = edge_index[1].reshape(NS, NCH2, CH)
    ew16 = edge_attr.reshape(NS, NCH2, CH)
    dst32 = edge_index[1].reshape(NT, NCH, CH)
    ew32 = edge_attr.reshape(NT, NCH, CH)

    def _prep(wih, bih, bhh):
        wt = wih.T
        wt = jnp.concatenate([wt[:, 0:NH], wt[:, 2 * NH:4 * NH]], axis=1)
        bb = (bih + bhh)
        bb = jnp.concatenate([bb[0:NH], bb[2 * NH:4 * NH]]).reshape(1, 3 * NH)
        return wt, bb

    wih1t, bi1 = _prep(Wih1, bih1, bhh1)
    wih2t, bi2 = _prep(Wih2, bih2, bhh2)

    _deg_kernel, _edge_scatter = _sc_kernels()
    degp = _deg_kernel(dst32, ew32)
    xw1, xsum = _mm_sum(x, W1)
    ys1, disr = _dis_scale(degp, xw1)
    part1 = _edge_scatter(ys1, src, dst16, ew16)
    h1, st1 = _gcn_post(part1, ys1, disr, b1.reshape(1, NH))
    h1n, ys2 = _bn_mm(h1, st1, g1.reshape(1, NH), be1.reshape(1, NH),
                      W2, disr)
    part2 = _edge_scatter(ys2, src, dst16, ew16)
    h2, st2 = _gcn_post(part2, ys2, disr, b2.reshape(1, NH))
    out = _head(h1n, h2, st2, g2.reshape(1, NH), be2.reshape(1, NH),
                wih1t, bi1, wih2t, bi2,
                Wfc1, bfc1.reshape(1, NH), Wfc2, bfc2.reshape(1, 1),
                xsum / N)
    return out.reshape(N)
